# baseline (speedup 1.0000x reference)
import functools

import jax
import jax.numpy as jnp
from jax import lax
from jax.experimental import pallas as pl
from jax.experimental.pallas import tpu as pltpu
from jax.experimental.pallas import tpu_sc as plsc

MODEL = 64
SCALE = 8.0
NC = 2
NS = 16
LANES = 16
NW = NC * NS
CHUNK = 400
NBUF = 4


def _scale_chunk(rows_ref):
    def body(i, carry):
        r = i * 4
        for rr in range(4):
            for j in range(MODEL // LANES):
                sl = pl.ds(j * LANES, LANES)
                rows_ref[r + rr, sl] = rows_ref[r + rr, sl] * SCALE
        return carry
    lax.fori_loop(0, CHUNK // 4, body, 0)


@functools.lru_cache(maxsize=None)
def _make_kernel(B):
    b_per_w = B // NW
    n_chunks = b_per_w // CHUNK
    n_rounds = n_chunks // NBUF
    mesh = plsc.VectorSubcoreMesh(core_axis_name="c", subcore_axis_name="s")

    @functools.partial(
        pl.kernel,
        mesh=mesh,
        compiler_params=pltpu.CompilerParams(use_tc_tiling_on_sc=False),
        out_type=jax.ShapeDtypeStruct((B, MODEL), jnp.float32),
        scratch_types=(
            [pltpu.VMEM((CHUNK,), jnp.int32) for _ in range(NBUF)]
            + [pltpu.VMEM((CHUNK, MODEL), jnp.float32) for _ in range(NBUF)]
            + [pltpu.SemaphoreType.DMA for _ in range(2 * NBUF)]
        ),
    )
    def k(table_hbm, idx_hbm, out_hbm, *scratch):
        idxs = scratch[0:NBUF]
        rows = scratch[NBUF:2 * NBUF]
        sem_g = scratch[2 * NBUF:3 * NBUF]
        sem_o = scratch[3 * NBUF:4 * NBUF]

        wid = lax.axis_index("s") * NC + lax.axis_index("c")
        wbase = wid * b_per_w

        def fetch(g, b):
            base = wbase + g * CHUNK
            pltpu.sync_copy(idx_hbm.at[pl.ds(base, CHUNK)], idxs[b])
            pltpu.async_copy(table_hbm.at[idxs[b]], rows[b], sem_g[b])

        def wait_gather(b):
            pltpu.make_async_copy(
                table_hbm.at[idxs[b]], rows[b], sem_g[b]).wait()

        def start_out(g, b):
            base = wbase + g * CHUNK
            pltpu.async_copy(rows[b], out_hbm.at[pl.ds(base, CHUNK)], sem_o[b])

        def wait_out(g, b):
            base = wbase + g * CHUNK
            pltpu.make_async_copy(
                rows[b], out_hbm.at[pl.ds(base, CHUNK)], sem_o[b]).wait()

        def process(g, b, prefetch, prefetch_wait_out):
            wait_gather(b)
            _scale_chunk(rows[b])
            start_out(g, b)
            if prefetch:
                bp = (b + 2) % NBUF
                if prefetch_wait_out:
                    wait_out(g, bp)
                fetch(g + 2, bp)

        fetch(0, 0)
        fetch(1, 1)

        for b in range(NBUF):
            process(b, b, prefetch=True, prefetch_wait_out=(b >= 2))

        def round_body(r, carry):
            g0 = r * NBUF
            for b in range(NBUF):
                process(g0 + b, b, prefetch=True, prefetch_wait_out=True)
            return carry
        lax.fori_loop(1, n_rounds - 1, round_body, 0)

        g0 = (n_rounds - 1) * NBUF
        for b in range(NBUF):
            process(g0 + b, b, prefetch=(b < 2), prefetch_wait_out=True)
        for b in range(NBUF):
            wait_out(g0 + b, b)

    return k


def kernel(x, table):
    bsz, hist = x.shape
    B = bsz * hist
    k = _make_kernel(B)
    out = k(table, x.reshape(B))
    return out.reshape(bsz, hist, MODEL)


# speedup vs baseline: 3.0482x; 3.0482x over previous
import functools

import jax
import jax.numpy as jnp
from jax import lax
from jax.experimental import pallas as pl
from jax.experimental.pallas import tpu as pltpu
from jax.experimental.pallas import tpu_sc as plsc

MODEL = 64
SCALE = 8.0
NC = 2
NS = 16
NW = NC * NS
LANES = 16

NH = 50
NB = 16384
CH = 256
CHUNKS_PER_H = NB // CH
TOTAL_CHUNKS = NH * CHUNKS_PER_H
PER_W = TOTAL_CHUNKS // NW
NBUF = 3
OUT_ELEMS = NH * 8 * (NB // 128) * 8 * 128
BLOCK = 8 * 2 * 8 * 128


VOCAB = 1000000
VCH = 256
NSLAB = 999936 // VCH
TAILV = VOCAB - NSLAB * VCH


@functools.lru_cache(maxsize=None)
def _make_transpose_kernel():
    mesh = plsc.VectorSubcoreMesh(core_axis_name="c", subcore_axis_name="s")

    @functools.partial(
        pl.kernel,
        mesh=mesh,
        compiler_params=pltpu.CompilerParams(
            use_tc_tiling_on_sc=True, needs_layout_passes=False),
        out_type=jax.ShapeDtypeStruct((VOCAB * MODEL,), jnp.float32),
        scratch_types=(
            [pltpu.VMEM((MODEL, VCH), jnp.float32) for _ in range(2)]
            + [pltpu.VMEM((VCH * 65,), jnp.float32)]
            + [pltpu.VMEM((VCH * MODEL,), jnp.float32) for _ in range(2)]
            + [pltpu.VMEM((MODEL, TAILV), jnp.float32)]
            + [pltpu.SemaphoreType.DMA for _ in range(4)]
        ),
    )
    def k(tabt_hbm, tail_hbm, out_hbm, *scratch):
        inb = scratch[0:2]
        opad = scratch[2]
        outb = scratch[3:5]
        tailb = scratch[5]
        sem_i = scratch[6:8]
        sem_o = scratch[8:10]

        wid = lax.axis_index("s") * NC + lax.axis_index("c")
        iota = lax.iota(jnp.int32, LANES)
        ivecs = [lax.mul(iota + vg * LANES, 65) for vg in range(VCH // LANES)]

        def fetch(c, s):
            pltpu.async_copy(
                tabt_hbm.at[:, pl.ds(c * VCH, VCH)], inb[s], sem_i[s])

        def wait_in(c, s):
            pltpu.make_async_copy(
                tabt_hbm.at[:, pl.ds(c * VCH, VCH)], inb[s], sem_i[s]).wait()

        def start_out(c, s):
            pltpu.async_copy(
                outb[s], out_hbm.at[pl.ds(c * (VCH * MODEL), VCH * MODEL)],
                sem_o[s])

        def wait_out(c, s):
            pltpu.make_async_copy(
                outb[s], out_hbm.at[pl.ds(c * (VCH * MODEL), VCH * MODEL)],
                sem_o[s]).wait()

        def transpose_slab(s, nvg, src=None):
            src = inb[s] if src is None else src

            @plsc.parallel_loop(0, MODEL, unroll=2)
            def _scatter(d):
                for vg in range(nvg):
                    v = src[d, pl.ds(vg * LANES, LANES)]
                    plsc.store_scatter(opad, [ivecs[vg] + d], v)

            @plsc.parallel_loop(0, nvg * LANES, unroll=4)
            def _repack(r):
                so = lax.mul(r, 65)
                do = lax.mul(r, MODEL)
                for w in range(MODEL // LANES):
                    outb[s][pl.ds(do + w * LANES, LANES)] = (
                        opad[pl.ds(so + w * LANES, LANES)])

        fetch(wid, 0)

        def round_body(r, carry):
            for s in range(2):
                t = r * 2 + s
                c = wid + t * NW

                @pl.when(c + NW < NSLAB)
                def _():
                    fetch(c + NW, 1 - s)

                @pl.when(c < NSLAB)
                def _():
                    wait_in(c, s)

                    @pl.when(t >= 2)
                    def _():
                        wait_out(c - 2 * NW, s)

                    transpose_slab(s, VCH // LANES)
                    start_out(c, s)

            return carry

        lax.fori_loop(0, 61, round_body, 0)

        wait_out(wid + 120 * NW, 0)
        c122 = wid + 122 * NW

        @pl.when(c122 < NSLAB)
        def _():
            wait_in(c122, 0)
            transpose_slab(0, VCH // LANES)
            start_out(c122, 0)

        wait_out(wid + 121 * NW, 1)

        @pl.when(c122 < NSLAB)
        def _():
            wait_out(c122, 0)

        @pl.when(wid == 0)
        def _():
            pltpu.sync_copy(tail_hbm, tailb)
            transpose_slab(0, TAILV // LANES, src=tailb)
            pltpu.sync_copy(
                outb[0].at[pl.ds(0, TAILV * MODEL)],
                out_hbm.at[pl.ds(NSLAB * VCH * MODEL, TAILV * MODEL)])

    return k


@functools.lru_cache(maxsize=None)
def _make_kernel():
    mesh = plsc.VectorSubcoreMesh(core_axis_name="c", subcore_axis_name="s")

    @functools.partial(
        pl.kernel,
        mesh=mesh,
        compiler_params=pltpu.CompilerParams(
            use_tc_tiling_on_sc=False, needs_layout_passes=False),
        out_type=jax.ShapeDtypeStruct((OUT_ELEMS,), jnp.float32),
        scratch_types=(
            [pltpu.VMEM((CH,), jnp.int32) for _ in range(NBUF)]
            + [pltpu.VMEM((CH, MODEL), jnp.float32) for _ in range(NBUF)]
            + [pltpu.VMEM((BLOCK,), jnp.float32) for _ in range(NBUF)]
            + [pltpu.VMEM((8 * 2072,), jnp.float32)]
            + [pltpu.SemaphoreType.DMA for _ in range(2 * NBUF)]
        ),
    )
    def k(table_hbm, idx_hbm, out_hbm, *scratch):
        idxb = scratch[0:NBUF]
        rows = scratch[NBUF:2 * NBUF]
        obuf = scratch[2 * NBUF:3 * NBUF]
        opad = scratch[3 * NBUF]
        sem_g = scratch[3 * NBUF + 1:4 * NBUF + 1]
        sem_o = scratch[4 * NBUF + 1:5 * NBUF + 1]

        wid = lax.axis_index("s") * NC + lax.axis_index("c")
        c0 = wid * PER_W
        iota = lax.iota(jnp.int32, LANES)
        flatq = [
            lax.mul(lax.shift_right_logical(iota, 3) + q * 2, 2072)
            + lax.mul(iota & 7, 129)
            for q in range(MODEL // LANES)
        ]

        def fetch(g, s):
            c = c0 + g
            h = c // CHUNKS_PER_H
            r = c % CHUNKS_PER_H
            start = h * NB + r * CH
            pltpu.sync_copy(idx_hbm.at[pl.ds(start, CH)], idxb[s])
            pltpu.async_copy(table_hbm.at[idxb[s]], rows[s], sem_g[s])

        def wait_gather(s):
            pltpu.make_async_copy(
                table_hbm.at[idxb[s]], rows[s], sem_g[s]).wait()

        def start_out(g, s):
            c = c0 + g
            h = c // CHUNKS_PER_H
            r = c % CHUNKS_PER_H
            for d0 in range(8):
                base = h * (8 * NB * 8) + d0 * (NB // 128 * 1024) + r * 2048
                pltpu.async_copy(
                    obuf[s].at[pl.ds(d0 * 2048, 2048)],
                    out_hbm.at[pl.ds(base, 2048)],
                    sem_o[s],
                )

        def wait_out(s):
            for d0 in range(8):
                pltpu.make_async_copy(
                    obuf[s].at[pl.ds(d0 * 2048, 2048)],
                    out_hbm.at[pl.ds(d0 * 2048, 2048)],
                    sem_o[s],
                ).wait()

        def transpose_scale(s):
            for p in range(2):
                pbase = p * 1032

                @plsc.parallel_loop(0, CH // 2, unroll=4)
                def _body(j1):
                    j = p * (CH // 2) + j1
                    sb = pbase + j1
                    for q in range(MODEL // LANES):
                        v = rows[s][j, pl.ds(q * LANES, LANES)]
                        plsc.store_scatter(opad, [flatq[q] + sb], v * SCALE)

            @plsc.parallel_loop(0, 128, unroll=4)
            def _repack(rr):
                so = (
                    lax.mul(lax.shift_right_logical(rr, 4), 2072)
                    + lax.mul(lax.shift_right_logical(rr, 3) & 1, 1032)
                    + lax.mul(rr & 7, 129)
                )
                do = lax.mul(rr, 128)
                for w in range(8):
                    obuf[s][pl.ds(do + w * LANES, LANES)] = (
                        opad[pl.ds(so + w * LANES, LANES)])

        fetch(0, 0)
        fetch(1, 1)

        for g in range(3):
            if g + 2 < PER_W:
                fetch(g + 2, (g + 2) % NBUF)
            wait_gather(g % NBUF)
            transpose_scale(g % NBUF)
            start_out(g, g % NBUF)

        def round_body(r, carry):
            for kk in range(NBUF):
                g = r * NBUF + kk
                s = kk
                gt = g + 2

                @pl.when(gt < PER_W)
                def _():
                    fetch(gt, (kk + 2) % NBUF)

                wait_gather(s)
                wait_out(s)
                transpose_scale(s)
                start_out(g, s)
            return carry

        lax.fori_loop(1, PER_W // NBUF, round_body, 0)

        g = PER_W - 1
        s = g % NBUF
        wait_gather(s)
        wait_out(s)
        transpose_scale(s)
        start_out(g, s)

        for g in range(PER_W - NBUF, PER_W):
            wait_out(g % NBUF)

    return k


def kernel(x, table):
    idx_t = x.T.reshape(x.shape[0] * x.shape[1])
    tabt = table.T
    dense = _make_transpose_kernel()(
        tabt, tabt[:, NSLAB * VCH:]).reshape(VOCAB, MODEL)
    k = _make_kernel()
    flat = k(dense, idx_t)
    a6 = flat.reshape(NH, 8, NB // 128, 8, 128)
    out = jnp.transpose(a6, (2, 4, 0, 1, 3)).reshape(
        x.shape[0], x.shape[1], MODEL)
    return out
